# Initial kernel scaffold; baseline (speedup 1.0000x reference)
import jax
import jax.numpy as jnp
from jax.experimental import pallas as pl


def kernel(z, W):
    raise NotImplementedError("write your pallas kernel here")


# speedup vs baseline: 1.3329x; 1.3329x over previous
import jax
import jax.numpy as jnp
from jax.experimental import pallas as pl


def _vq_kernel(z_ref, w_ref, out_ref):
    zt = z_ref[0]
    w = w_ref[...]
    xw = jax.lax.dot_general(
        w, zt, (((1,), (0,)), ((), ())),
        preferred_element_type=jnp.float32,
        precision=jax.lax.Precision.DEFAULT)
    w2 = jnp.sum(w * w, axis=1)
    x2 = jnp.sum(zt * zt, axis=0)
    d2 = (x2[None, :] + w2[:, None]) - 2.0 * xw
    out_ref[0, 0, :] = jnp.argmin(d2, axis=0).astype(jnp.int32)


def kernel(z, W):
    t, a, b, c = z.shape
    n = b * c
    k = W.shape[0]
    z3 = z.reshape(t, a, n)
    out = pl.pallas_call(
        _vq_kernel,
        grid=(t,),
        in_specs=[
            pl.BlockSpec((1, a, n), lambda i: (i, 0, 0)),
            pl.BlockSpec((k, a), lambda i: (0, 0)),
        ],
        out_specs=pl.BlockSpec((1, 1, n), lambda i: (i, 0, 0)),
        out_shape=jax.ShapeDtypeStruct((t, 1, n), jnp.int32),
    )(z3, W)
    return out.reshape(t, b, c)
